# baseline (speedup 1.0000x reference)
import functools

import numpy as np

import jax
import jax.numpy as jnp
from jax import lax
from jax.experimental import pallas as pl
from jax.experimental.pallas import tpu as pltpu
from jax.experimental.pallas import tpu_sc as plsc

MP_ = 2
N_ = 10000
E_ = 320000
D_ = 128
H_ = 128

NC_ = 2
NS_ = 16
NW_ = NC_ * NS_
EPW_ = E_ // NW_
CHUNK_ = 80
NCHUNK_ = EPW_ // CHUNK_
NPAD_ = 10240


def _sc_mesh():
    return plsc.VectorSubcoreMesh(core_axis_name="c", subcore_axis_name="s")


def _gather_add_sc(tab0, tab1, idx0, idx1):
    @functools.partial(
        pl.kernel,
        out_type=jax.ShapeDtypeStruct((E_, H_), jnp.float32),
        mesh=_sc_mesh(),
        scratch_types=[
            pltpu.VMEM((NCHUNK_, CHUNK_), jnp.int32),
            pltpu.VMEM((NCHUNK_, CHUNK_), jnp.int32),
            pltpu.VMEM((2, CHUNK_, H_), jnp.float32),
            pltpu.VMEM((2, CHUNK_, H_), jnp.float32),
            pltpu.VMEM((2, CHUNK_, H_), jnp.float32),
            pltpu.SemaphoreType.DMA,
            pltpu.SemaphoreType.DMA,
            pltpu.SemaphoreType.DMA,
            pltpu.SemaphoreType.DMA,
            pltpu.SemaphoreType.DMA,
            pltpu.SemaphoreType.DMA,
        ],
    )
    def k(tab0_hbm, tab1_hbm, idx0_hbm, idx1_hbm, out_hbm,
          idx0_v, idx1_v, bp, bq, bo, gp0, gp1, gq0, gq1, w0, w1):
        wid = lax.axis_index("s") * NC_ + lax.axis_index("c")
        pltpu.sync_copy(idx0_hbm.at[wid], idx0_v)
        pltpu.sync_copy(idx1_hbm.at[wid], idx1_v)
        base = wid * EPW_
        gsems = (gp0, gp1)
        qsems = (gq0, gq1)
        wsems = (w0, w1)

        def start_g(j, slot):
            pltpu.async_copy(tab0_hbm.at[idx0_v.at[j]], bp.at[slot], gsems[slot])
            pltpu.async_copy(tab1_hbm.at[idx1_v.at[j]], bq.at[slot], qsems[slot])

        def wait_g(slot):
            pltpu.make_async_copy(
                tab0_hbm.at[idx0_v.at[0]], bp.at[slot], gsems[slot]).wait()
            pltpu.make_async_copy(
                tab1_hbm.at[idx1_v.at[0]], bq.at[slot], qsems[slot]).wait()

        def start_w(j, slot):
            pltpu.async_copy(
                bo.at[slot], out_hbm.at[pl.ds(base + j * CHUNK_, CHUNK_)],
                wsems[slot])

        def wait_w(slot):
            pltpu.make_async_copy(
                bo.at[slot], out_hbm.at[pl.ds(base, CHUNK_)], wsems[slot]).wait()

        def compute(slot):
            bp_s, bq_s, bo_s = bp.at[slot], bq.at[slot], bo.at[slot]

            def row(r, carry):
                for c in range(8):
                    bo_s[r, pl.ds(16 * c, 16)] = (
                        bp_s[r, pl.ds(16 * c, 16)] + bq_s[r, pl.ds(16 * c, 16)])
                return carry

            lax.fori_loop(0, CHUNK_, row, 0)

        start_g(0, 0)

        def pair(k_, carry):
            j0 = 2 * k_
            start_g(j0 + 1, 1)
            wait_g(0)

            @pl.when(k_ > 0)
            def _():
                wait_w(0)

            compute(0)
            start_w(j0, 0)
            start_g(j0 + 2, 0)
            wait_g(1)

            @pl.when(k_ > 0)
            def _():
                wait_w(1)

            compute(1)
            start_w(j0 + 1, 1)
            return carry

        lax.fori_loop(0, (NCHUNK_ - 1) // 2, pair, 0)
        wait_g(0)
        wait_w(0)
        compute(0)
        start_w(NCHUNK_ - 1, 0)
        wait_w(0)
        wait_w(1)

    return k(tab0, tab1, idx0, idx1)


def _scatter_sc(ea, idx1, zinit):
    @functools.partial(
        pl.kernel,
        out_type=jax.ShapeDtypeStruct((NC_, NPAD_, D_), jnp.float32),
        mesh=_sc_mesh(),
        scratch_types=[
            pltpu.VMEM((NCHUNK_, CHUNK_), jnp.int32),
            pltpu.VMEM((2, CHUNK_, D_), jnp.float32),
            pltpu.VMEM_SHARED((NPAD_, D_), jnp.float32),
            pltpu.SemaphoreType.DMA,
            pltpu.SemaphoreType.DMA,
            pltpu.SemaphoreType.DMA,
            pltpu.SemaphoreType.DMA,
        ],
    )
    def k(ea_hbm, idx_hbm, z_hbm, out_hbm, idx_v, buf, acc_sh, l0, l1, a0, a1):
        c = lax.axis_index("c")
        s = lax.axis_index("s")
        wid = s * NC_ + c
        rows_per_s = NPAD_ // NS_
        pltpu.sync_copy(z_hbm.at[pl.ds(s * rows_per_s, rows_per_s)],
                        acc_sh.at[pl.ds(s * rows_per_s, rows_per_s)])
        pltpu.sync_copy(idx_hbm.at[wid], idx_v)
        plsc.subcore_barrier()
        base = wid * EPW_
        lsems = (l0, l1)
        asems = (a0, a1)

        def start_l(j, slot):
            pltpu.async_copy(ea_hbm.at[pl.ds(base + j * CHUNK_, CHUNK_)],
                             buf.at[slot], lsems[slot])

        def wait_l(slot):
            pltpu.make_async_copy(ea_hbm.at[pl.ds(base, CHUNK_)],
                                  buf.at[slot], lsems[slot]).wait()

        def start_a(j, slot):
            pltpu.async_copy(buf.at[slot], acc_sh.at[idx_v.at[j]],
                             asems[slot], add=True)

        def wait_a(slot):
            pltpu.make_async_copy(buf.at[slot], acc_sh.at[idx_v.at[0]],
                                  asems[slot]).wait()

        start_l(0, 0)

        def pair(k_, carry):
            j0 = 2 * k_

            @pl.when(k_ > 0)
            def _():
                wait_a(1)

            start_l(j0 + 1, 1)
            wait_l(0)
            start_a(j0, 0)
            wait_a(0)
            start_l(j0 + 2, 0)
            wait_l(1)
            start_a(j0 + 1, 1)
            return carry

        lax.fori_loop(0, (NCHUNK_ - 1) // 2, pair, 0)
        wait_a(1)
        wait_l(0)
        start_a(NCHUNK_ - 1, 0)
        wait_a(0)
        plsc.subcore_barrier()
        pltpu.sync_copy(acc_sh.at[pl.ds(s * rows_per_s, rows_per_s)],
                        out_hbm.at[c].at[pl.ds(s * rows_per_s, rows_per_s)])

    return k(ea, idx1, zinit)


def _prep_tc(x, w1a, w1b):
    BN = 2000
    grid = (N_ // BN,)

    def body(x_ref, wa_ref, wb_ref, p_ref, q_ref):
        xb = x_ref[...].astype(jnp.bfloat16)
        p_ref[...] = jnp.dot(xb, wa_ref[...], preferred_element_type=jnp.float32)
        q_ref[...] = jnp.dot(xb, wb_ref[...], preferred_element_type=jnp.float32)

    row = pl.BlockSpec((BN, D_), lambda i: (i, 0))
    w = pl.BlockSpec((D_, H_), lambda i: (0, 0))
    return pl.pallas_call(
        body, grid=grid,
        in_specs=[row, w, w],
        out_specs=[pl.BlockSpec((BN, H_), lambda i: (i, 0))] * 2,
        out_shape=[jax.ShapeDtypeStruct((N_, H_), jnp.float32)] * 2,
    )(x, w1a, w1b)


def _mlp_tail(h, w2, b2, w3cat, b3cat, g, bb):
    h = jnp.maximum(
        jnp.dot(h.astype(jnp.bfloat16), w2, preferred_element_type=jnp.float32)
        + b2, 0.0)
    t = jnp.dot(h.astype(jnp.bfloat16), w3cat,
                preferred_element_type=jnp.float32) + b3cat
    h = t[:, :H_]
    mu = t[:, H_:]
    d = h - mu
    var = jnp.mean(d * d, axis=-1, keepdims=True)
    return d * lax.rsqrt(var + 1e-5) * g + bb


def _ln_weights(w3, b3):
    w3m = jnp.tile(jnp.sum(w3, axis=1, keepdims=True) / H_, (1, H_))
    w3cat = jnp.concatenate([w3, w3m], axis=1).astype(jnp.bfloat16)
    b3cat = jnp.concatenate(
        [b3, jnp.full((H_,), jnp.mean(b3), jnp.float32)]).reshape(1, 2 * H_)
    return w3cat, b3cat


def _edge_mlp_tc(gsum, ea, w1c, b1, w2, b2, w3cat, b3cat, g, bb):
    BE = 1280
    grid = (E_ // BE,)

    def body(gs_ref, ea_ref, w1_ref, b1_ref, w2_ref, b2_ref,
             w3_ref, b3_ref, g_ref, bb_ref, out_ref):
        ea_b = ea_ref[...]
        h = (gs_ref[...] + b1_ref[...]
             + jnp.dot(ea_b.astype(jnp.bfloat16), w1_ref[...],
                       preferred_element_type=jnp.float32))
        h = jnp.maximum(h, 0.0)
        out_ref[...] = _mlp_tail(h, w2_ref[...], b2_ref[...], w3_ref[...],
                                 b3_ref[...], g_ref[...], bb_ref[...]) + ea_b

    row = pl.BlockSpec((BE, H_), lambda i: (i, 0))
    w = pl.BlockSpec((H_, H_), lambda i: (0, 0))
    wcat = pl.BlockSpec((H_, 2 * H_), lambda i: (0, 0))
    b = pl.BlockSpec((1, H_), lambda i: (0, 0))
    bcat = pl.BlockSpec((1, 2 * H_), lambda i: (0, 0))
    return pl.pallas_call(
        body, grid=grid,
        in_specs=[row, row, w, b, w, b, wcat, bcat, b, b],
        out_specs=pl.BlockSpec((BE, D_), lambda i: (i, 0)),
        out_shape=jax.ShapeDtypeStruct((E_, D_), jnp.float32),
    )(gsum, ea, w1c, b1.reshape(1, -1), w2, b2.reshape(1, -1),
      w3cat, b3cat, g.reshape(1, -1), bb.reshape(1, -1))


def _node_mlp_tc(x, parts, w1a, w1b, b1, w2, b2, w3cat, b3cat, g, bb):
    BN = 2000
    grid = (N_ // BN,)

    def body(x_ref, p_ref, w1a_ref, w1b_ref, b1_ref, w2_ref, b2_ref,
             w3_ref, b3_ref, g_ref, bb_ref, out_ref):
        xb = x_ref[...]
        agg = p_ref[0] + p_ref[1]
        h = (jnp.dot(xb.astype(jnp.bfloat16), w1a_ref[...],
                     preferred_element_type=jnp.float32)
             + jnp.dot(agg.astype(jnp.bfloat16), w1b_ref[...],
                       preferred_element_type=jnp.float32)
             + b1_ref[...])
        h = jnp.maximum(h, 0.0)
        out_ref[...] = _mlp_tail(h, w2_ref[...], b2_ref[...], w3_ref[...],
                                 b3_ref[...], g_ref[...], bb_ref[...]) + xb

    row = pl.BlockSpec((BN, D_), lambda i: (i, 0))
    prow = pl.BlockSpec((NC_, BN, D_), lambda i: (0, i, 0))
    w = pl.BlockSpec((D_, H_), lambda i: (0, 0))
    wcat = pl.BlockSpec((H_, 2 * H_), lambda i: (0, 0))
    b = pl.BlockSpec((1, H_), lambda i: (0, 0))
    bcat = pl.BlockSpec((1, 2 * H_), lambda i: (0, 0))
    return pl.pallas_call(
        body, grid=grid,
        in_specs=[row, prow, w, w, b, w, b, wcat, bcat, b, b],
        out_specs=row,
        out_shape=jax.ShapeDtypeStruct((N_, D_), jnp.float32),
    )(x, parts, w1a, w1b, b1.reshape(1, -1), w2, b2.reshape(1, -1),
      w3cat, b3cat, g.reshape(1, -1), bb.reshape(1, -1))


def kernel(x, edge_indices, edge_attrs, eW1, eb1, eW2, eb2, eW3, eb3, eg, ebb,
           nW1, nb1, nW2, nb2, nW3, nb3, ng, nbb):
    ei = edge_indices[0].astype(jnp.int32)
    idx0 = ei[0].reshape(NW_, NCHUNK_, CHUNK_)
    idx1 = ei[1].reshape(NW_, NCHUNK_, CHUNK_)
    ea = edge_attrs[0]
    zinit = jnp.zeros((NPAD_, D_), jnp.float32)

    eW1h = eW1.astype(jnp.bfloat16)
    eW2h = eW2.astype(jnp.bfloat16)
    nW1h = nW1.astype(jnp.bfloat16)
    nW2h = nW2.astype(jnp.bfloat16)

    for i in range(MP_):
        p_tab, q_tab = _prep_tc(x, eW1h[i, :D_], eW1h[i, D_:2 * D_])
        gsum = _gather_add_sc(p_tab, q_tab, idx0, idx1)
        ew3cat, eb3cat = _ln_weights(eW3[i], eb3[i])
        ea = _edge_mlp_tc(gsum, ea, eW1h[i, 2 * D_:], eb1[i], eW2h[i], eb2[i],
                          ew3cat, eb3cat, eg[i], ebb[i])
        parts = _scatter_sc(ea, idx1, zinit)
        nw3cat, nb3cat = _ln_weights(nW3[i], nb3[i])
        x = _node_mlp_tc(x, parts, nW1h[i, :D_], nW1h[i, D_:], nb1[i],
                         nW2h[i], nb2[i], nw3cat, nb3cat, ng[i], nbb[i])
    return (x, ea[None])


# speedup vs baseline: 1.0084x; 1.0084x over previous
import functools

import numpy as np

import jax
import jax.numpy as jnp
from jax import lax
from jax.experimental import pallas as pl
from jax.experimental.pallas import tpu as pltpu
from jax.experimental.pallas import tpu_sc as plsc

MP_ = 2
N_ = 10000
E_ = 320000
D_ = 128
H_ = 128

NC_ = 2
NS_ = 16
NW_ = NC_ * NS_
EPW_ = E_ // NW_
CHUNK_ = 80
NCHUNK_ = EPW_ // CHUNK_
NPAD_ = 10240


def _sc_mesh():
    return plsc.VectorSubcoreMesh(core_axis_name="c", subcore_axis_name="s")


def _gather_add_sc(tab0, tab1, idx0, idx1, e_tot):
    epw = e_tot // NW_
    nchunk = epw // CHUNK_

    @functools.partial(
        pl.kernel,
        out_type=jax.ShapeDtypeStruct((e_tot, H_), jnp.float32),
        mesh=_sc_mesh(),
        scratch_types=[
            pltpu.VMEM((nchunk, CHUNK_), jnp.int32),
            pltpu.VMEM((nchunk, CHUNK_), jnp.int32),
            pltpu.VMEM((2, CHUNK_, H_), jnp.float32),
            pltpu.VMEM((2, CHUNK_, H_), jnp.float32),
            pltpu.VMEM((2, CHUNK_, H_), jnp.float32),
            pltpu.SemaphoreType.DMA,
            pltpu.SemaphoreType.DMA,
            pltpu.SemaphoreType.DMA,
            pltpu.SemaphoreType.DMA,
            pltpu.SemaphoreType.DMA,
            pltpu.SemaphoreType.DMA,
        ],
    )
    def k(tab0_hbm, tab1_hbm, idx0_hbm, idx1_hbm, out_hbm,
          idx0_v, idx1_v, bp, bq, bo, gp0, gp1, gq0, gq1, w0, w1):
        wid = lax.axis_index("s") * NC_ + lax.axis_index("c")
        pltpu.sync_copy(idx0_hbm.at[wid], idx0_v)
        pltpu.sync_copy(idx1_hbm.at[wid], idx1_v)
        base = wid * epw
        gsems = (gp0, gp1)
        qsems = (gq0, gq1)
        wsems = (w0, w1)

        def start_g(j, slot):
            pltpu.async_copy(tab0_hbm.at[idx0_v.at[j]], bp.at[slot], gsems[slot])
            pltpu.async_copy(tab1_hbm.at[idx1_v.at[j]], bq.at[slot], qsems[slot])

        def wait_g(slot):
            pltpu.make_async_copy(
                tab0_hbm.at[idx0_v.at[0]], bp.at[slot], gsems[slot]).wait()
            pltpu.make_async_copy(
                tab1_hbm.at[idx1_v.at[0]], bq.at[slot], qsems[slot]).wait()

        def start_w(j, slot):
            pltpu.async_copy(
                bo.at[slot], out_hbm.at[pl.ds(base + j * CHUNK_, CHUNK_)],
                wsems[slot])

        def wait_w(slot):
            pltpu.make_async_copy(
                bo.at[slot], out_hbm.at[pl.ds(base, CHUNK_)], wsems[slot]).wait()

        def compute(slot):
            bp_s, bq_s, bo_s = bp.at[slot], bq.at[slot], bo.at[slot]

            def row(r, carry):
                for c in range(8):
                    bo_s[r, pl.ds(16 * c, 16)] = (
                        bp_s[r, pl.ds(16 * c, 16)] + bq_s[r, pl.ds(16 * c, 16)])
                return carry

            lax.fori_loop(0, CHUNK_, row, 0)

        start_g(0, 0)
        npairs = nchunk // 2

        def pair(k_, carry):
            j0 = 2 * k_
            start_g(j0 + 1, 1)
            wait_g(0)

            @pl.when(k_ > 0)
            def _():
                wait_w(0)

            compute(0)
            start_w(j0, 0)

            @pl.when(j0 + 2 < nchunk)
            def _():
                start_g(j0 + 2, 0)

            wait_g(1)

            @pl.when(k_ > 0)
            def _():
                wait_w(1)

            compute(1)
            start_w(j0 + 1, 1)
            return carry

        lax.fori_loop(0, npairs, pair, 0)
        if nchunk % 2 == 1:
            wait_g(0)
            wait_w(0)
            compute(0)
            start_w(nchunk - 1, 0)
        wait_w(0)
        wait_w(1)

    return k(tab0, tab1, idx0, idx1)


def _scatter_sc(ea, idx1, zinit, e_tot):
    epw = e_tot // NW_
    nchunk = epw // CHUNK_

    @functools.partial(
        pl.kernel,
        out_type=jax.ShapeDtypeStruct((NC_, NPAD_, D_), jnp.float32),
        mesh=_sc_mesh(),
        scratch_types=[
            pltpu.VMEM((nchunk, CHUNK_), jnp.int32),
            pltpu.VMEM((2, CHUNK_, D_), jnp.float32),
            pltpu.VMEM_SHARED((NPAD_, D_), jnp.float32),
            pltpu.SemaphoreType.DMA,
            pltpu.SemaphoreType.DMA,
            pltpu.SemaphoreType.DMA,
            pltpu.SemaphoreType.DMA,
        ],
    )
    def k(ea_hbm, idx_hbm, z_hbm, out_hbm, idx_v, buf, acc_sh, l0, l1, a0, a1):
        c = lax.axis_index("c")
        s = lax.axis_index("s")
        wid = s * NC_ + c
        rows_per_s = NPAD_ // NS_
        pltpu.sync_copy(z_hbm.at[c].at[pl.ds(s * rows_per_s, rows_per_s)],
                        acc_sh.at[pl.ds(s * rows_per_s, rows_per_s)])
        pltpu.sync_copy(idx_hbm.at[wid], idx_v)
        plsc.subcore_barrier()
        base = wid * epw
        lsems = (l0, l1)
        asems = (a0, a1)

        def start_l(j, slot):
            pltpu.async_copy(ea_hbm.at[pl.ds(base + j * CHUNK_, CHUNK_)],
                             buf.at[slot], lsems[slot])

        def wait_l(slot):
            pltpu.make_async_copy(ea_hbm.at[pl.ds(base, CHUNK_)],
                                  buf.at[slot], lsems[slot]).wait()

        def start_a(j, slot):
            pltpu.async_copy(buf.at[slot], acc_sh.at[idx_v.at[j]],
                             asems[slot], add=True)

        def wait_a(slot):
            pltpu.make_async_copy(buf.at[slot], acc_sh.at[idx_v.at[0]],
                                  asems[slot]).wait()

        start_l(0, 0)
        npairs = nchunk // 2

        def pair(k_, carry):
            j0 = 2 * k_

            @pl.when(k_ > 0)
            def _():
                wait_a(1)

            start_l(j0 + 1, 1)
            wait_l(0)
            start_a(j0, 0)
            wait_a(0)

            @pl.when(j0 + 2 < nchunk)
            def _():
                start_l(j0 + 2, 0)

            wait_l(1)
            start_a(j0 + 1, 1)
            return carry

        lax.fori_loop(0, npairs, pair, 0)
        wait_a(1)
        if nchunk % 2 == 1:
            wait_l(0)
            start_a(nchunk - 1, 0)
            wait_a(0)
        plsc.subcore_barrier()
        pltpu.sync_copy(acc_sh.at[pl.ds(s * rows_per_s, rows_per_s)],
                        out_hbm.at[c].at[pl.ds(s * rows_per_s, rows_per_s)])

    return k(ea, idx1, zinit)


def _prep_tc(x, w1a, w1b):
    BN = 2000
    grid = (N_ // BN,)

    def body(x_ref, wa_ref, wb_ref, p_ref, q_ref):
        xb = x_ref[...].astype(jnp.bfloat16)
        p_ref[...] = jnp.dot(xb, wa_ref[...], preferred_element_type=jnp.float32)
        q_ref[...] = jnp.dot(xb, wb_ref[...], preferred_element_type=jnp.float32)

    row = pl.BlockSpec((BN, D_), lambda i: (i, 0))
    w = pl.BlockSpec((D_, H_), lambda i: (0, 0))
    return pl.pallas_call(
        body, grid=grid,
        in_specs=[row, w, w],
        out_specs=[pl.BlockSpec((BN, H_), lambda i: (i, 0))] * 2,
        out_shape=[jax.ShapeDtypeStruct((N_, H_), jnp.float32)] * 2,
    )(x, w1a, w1b)


def _mlp_tail(h, w2, b2, w3cat, b3cat, g, bb):
    h = jnp.maximum(
        jnp.dot(h.astype(jnp.bfloat16), w2, preferred_element_type=jnp.float32)
        + b2, 0.0)
    t = jnp.dot(h.astype(jnp.bfloat16), w3cat,
                preferred_element_type=jnp.float32) + b3cat
    h = t[:, :H_]
    mu = t[:, H_:]
    d = h - mu
    var = jnp.mean(d * d, axis=-1, keepdims=True)
    return d * lax.rsqrt(var + 1e-5) * g + bb


def _ln_weights(w3, b3):
    w3m = jnp.tile(jnp.sum(w3, axis=1, keepdims=True) / H_, (1, H_))
    w3cat = jnp.concatenate([w3, w3m], axis=1).astype(jnp.bfloat16)
    b3cat = jnp.concatenate(
        [b3, jnp.full((H_,), jnp.mean(b3), jnp.float32)]).reshape(1, 2 * H_)
    return w3cat, b3cat


def _edge_mlp_tc(gsum, ea, w1c, b1, w2, b2, w3cat, b3cat, g, bb, e_tot):
    BE = 1280
    grid = (e_tot // BE,)

    def body(gs_ref, ea_ref, w1_ref, b1_ref, w2_ref, b2_ref,
             w3_ref, b3_ref, g_ref, bb_ref, out_ref):
        ea_b = ea_ref[...]
        h = (gs_ref[...] + b1_ref[...]
             + jnp.dot(ea_b.astype(jnp.bfloat16), w1_ref[...],
                       preferred_element_type=jnp.float32))
        h = jnp.maximum(h, 0.0)
        out_ref[...] = _mlp_tail(h, w2_ref[...], b2_ref[...], w3_ref[...],
                                 b3_ref[...], g_ref[...], bb_ref[...]) + ea_b

    row = pl.BlockSpec((BE, H_), lambda i: (i, 0))
    w = pl.BlockSpec((H_, H_), lambda i: (0, 0))
    wcat = pl.BlockSpec((H_, 2 * H_), lambda i: (0, 0))
    b = pl.BlockSpec((1, H_), lambda i: (0, 0))
    bcat = pl.BlockSpec((1, 2 * H_), lambda i: (0, 0))
    return pl.pallas_call(
        body, grid=grid,
        in_specs=[row, row, w, b, w, b, wcat, bcat, b, b],
        out_specs=pl.BlockSpec((BE, D_), lambda i: (i, 0)),
        out_shape=jax.ShapeDtypeStruct((e_tot, D_), jnp.float32),
    )(gsum, ea, w1c, b1.reshape(1, -1), w2, b2.reshape(1, -1),
      w3cat, b3cat, g.reshape(1, -1), bb.reshape(1, -1))


def _node_mlp_tc(x, parts, w1a, w1b, b1, w2, b2, w3cat, b3cat, g, bb):
    BN = 2000
    grid = (N_ // BN,)

    def body(x_ref, p_ref, w1a_ref, w1b_ref, b1_ref, w2_ref, b2_ref,
             w3_ref, b3_ref, g_ref, bb_ref, out_ref):
        xb = x_ref[...]
        agg = p_ref[0] + p_ref[1]
        h = (jnp.dot(xb.astype(jnp.bfloat16), w1a_ref[...],
                     preferred_element_type=jnp.float32)
             + jnp.dot(agg.astype(jnp.bfloat16), w1b_ref[...],
                       preferred_element_type=jnp.float32)
             + b1_ref[...])
        h = jnp.maximum(h, 0.0)
        out_ref[...] = _mlp_tail(h, w2_ref[...], b2_ref[...], w3_ref[...],
                                 b3_ref[...], g_ref[...], bb_ref[...]) + xb

    row = pl.BlockSpec((BN, D_), lambda i: (i, 0))
    prow = pl.BlockSpec((NC_, BN, D_), lambda i: (0, i, 0))
    w = pl.BlockSpec((D_, H_), lambda i: (0, 0))
    wcat = pl.BlockSpec((H_, 2 * H_), lambda i: (0, 0))
    b = pl.BlockSpec((1, H_), lambda i: (0, 0))
    bcat = pl.BlockSpec((1, 2 * H_), lambda i: (0, 0))
    return pl.pallas_call(
        body, grid=grid,
        in_specs=[row, prow, w, w, b, w, b, wcat, bcat, b, b],
        out_specs=row,
        out_shape=jax.ShapeDtypeStruct((N_, D_), jnp.float32),
    )(x, parts, w1a, w1b, b1.reshape(1, -1), w2, b2.reshape(1, -1),
      w3cat, b3cat, g.reshape(1, -1), bb.reshape(1, -1))


EA_ = 153600
EB_ = E_ - EA_


def kernel(x, edge_indices, edge_attrs, eW1, eb1, eW2, eb2, eW3, eb3, eg, ebb,
           nW1, nb1, nW2, nb2, nW3, nb3, ng, nbb):
    ei = edge_indices[0].astype(jnp.int32)
    idx0a = ei[0, :EA_].reshape(NW_, -1, CHUNK_)
    idx0b = ei[0, EA_:].reshape(NW_, -1, CHUNK_)
    idx1a = ei[1, :EA_].reshape(NW_, -1, CHUNK_)
    idx1b = ei[1, EA_:].reshape(NW_, -1, CHUNK_)
    ea_a = edge_attrs[0, :EA_]
    ea_b = edge_attrs[0, EA_:]
    zinit = jnp.zeros((NC_, NPAD_, D_), jnp.float32)

    eW1h = eW1.astype(jnp.bfloat16)
    eW2h = eW2.astype(jnp.bfloat16)
    nW1h = nW1.astype(jnp.bfloat16)
    nW2h = nW2.astype(jnp.bfloat16)

    for i in range(MP_):
        p_tab, q_tab = _prep_tc(x, eW1h[i, :D_], eW1h[i, D_:2 * D_])
        ew3cat, eb3cat = _ln_weights(eW3[i], eb3[i])
        gsum_a = _gather_add_sc(p_tab, q_tab, idx0a, idx1a, EA_)
        gsum_b = _gather_add_sc(p_tab, q_tab, idx0b, idx1b, EB_)
        ea_a = _edge_mlp_tc(gsum_a, ea_a, eW1h[i, 2 * D_:], eb1[i], eW2h[i],
                            eb2[i], ew3cat, eb3cat, eg[i], ebb[i], EA_)
        parts = _scatter_sc(ea_a, idx1a, zinit, EA_)
        ea_b = _edge_mlp_tc(gsum_b, ea_b, eW1h[i, 2 * D_:], eb1[i], eW2h[i],
                            eb2[i], ew3cat, eb3cat, eg[i], ebb[i], EB_)
        parts = _scatter_sc(ea_b, idx1b, parts, EB_)
        nw3cat, nb3cat = _ln_weights(nW3[i], nb3[i])
        x = _node_mlp_tc(x, parts, nW1h[i, :D_], nW1h[i, D_:], nb1[i],
                         nW2h[i], nb2[i], nw3cat, nb3cat, ng[i], nbb[i])
    return (x, jnp.concatenate([ea_a, ea_b], axis=0)[None])


# speedup vs baseline: 1.1351x; 1.1256x over previous
import functools

import numpy as np

import jax
import jax.numpy as jnp
from jax import lax
from jax.experimental import pallas as pl
from jax.experimental.pallas import tpu as pltpu
from jax.experimental.pallas import tpu_sc as plsc

MP_ = 2
N_ = 10000
E_ = 320000
D_ = 128
H_ = 128

NC_ = 2
NS_ = 16
NW_ = NC_ * NS_
EPW_ = E_ // NW_
CHUNK_ = 80
NCHUNK_ = EPW_ // CHUNK_
NPAD_ = 10240


def _sc_mesh():
    return plsc.VectorSubcoreMesh(core_axis_name="c", subcore_axis_name="s")


def _gather_add_sc(tab0, tab1, idx0, idx1, e_tot):
    epw = e_tot // NW_
    nchunk = epw // CHUNK_

    @functools.partial(
        pl.kernel,
        out_type=jax.ShapeDtypeStruct((e_tot, H_), jnp.float32),
        mesh=_sc_mesh(),
        scratch_types=[
            pltpu.VMEM((nchunk, CHUNK_), jnp.int32),
            pltpu.VMEM((nchunk, CHUNK_), jnp.int32),
            pltpu.VMEM((2, CHUNK_, H_), jnp.float32),
            pltpu.VMEM((2, CHUNK_, H_), jnp.float32),
            pltpu.VMEM((2, CHUNK_, H_), jnp.float32),
            pltpu.SemaphoreType.DMA,
            pltpu.SemaphoreType.DMA,
            pltpu.SemaphoreType.DMA,
            pltpu.SemaphoreType.DMA,
            pltpu.SemaphoreType.DMA,
            pltpu.SemaphoreType.DMA,
        ],
    )
    def k(tab0_hbm, tab1_hbm, idx0_hbm, idx1_hbm, out_hbm,
          idx0_v, idx1_v, bp, bq, bo, gp0, gp1, gq0, gq1, w0, w1):
        wid = lax.axis_index("s") * NC_ + lax.axis_index("c")
        pltpu.sync_copy(idx0_hbm.at[wid], idx0_v)
        pltpu.sync_copy(idx1_hbm.at[wid], idx1_v)
        base = wid * epw
        gsems = (gp0, gp1)
        qsems = (gq0, gq1)
        wsems = (w0, w1)

        def start_g(j, slot):
            pltpu.async_copy(tab0_hbm.at[idx0_v.at[j]], bp.at[slot], gsems[slot])
            pltpu.async_copy(tab1_hbm.at[idx1_v.at[j]], bq.at[slot], qsems[slot])

        def wait_g(slot):
            pltpu.make_async_copy(
                tab0_hbm.at[idx0_v.at[0]], bp.at[slot], gsems[slot]).wait()
            pltpu.make_async_copy(
                tab1_hbm.at[idx1_v.at[0]], bq.at[slot], qsems[slot]).wait()

        def start_w(j, slot):
            pltpu.async_copy(
                bo.at[slot], out_hbm.at[pl.ds(base + j * CHUNK_, CHUNK_)],
                wsems[slot])

        def wait_w(slot):
            pltpu.make_async_copy(
                bo.at[slot], out_hbm.at[pl.ds(base, CHUNK_)], wsems[slot]).wait()

        def compute(slot):
            bp_s, bq_s, bo_s = bp.at[slot], bq.at[slot], bo.at[slot]

            def row(r, carry):
                for c in range(8):
                    bo_s[r, pl.ds(16 * c, 16)] = (
                        bp_s[r, pl.ds(16 * c, 16)] + bq_s[r, pl.ds(16 * c, 16)])
                return carry

            lax.fori_loop(0, CHUNK_, row, 0)

        start_g(0, 0)
        npairs = nchunk // 2

        def pair(k_, carry):
            j0 = 2 * k_
            start_g(j0 + 1, 1)
            wait_g(0)

            @pl.when(k_ > 0)
            def _():
                wait_w(0)

            compute(0)
            start_w(j0, 0)

            @pl.when(j0 + 2 < nchunk)
            def _():
                start_g(j0 + 2, 0)

            wait_g(1)

            @pl.when(k_ > 0)
            def _():
                wait_w(1)

            compute(1)
            start_w(j0 + 1, 1)
            return carry

        lax.fori_loop(0, npairs, pair, 0)
        if nchunk % 2 == 1:
            wait_g(0)
            wait_w(0)
            compute(0)
            start_w(nchunk - 1, 0)
        wait_w(0)
        wait_w(1)

    return k(tab0, tab1, idx0, idx1)


def _scatter_sc(ea, idx1, zinit, e_tot):
    epw = e_tot // NW_
    nchunk = epw // CHUNK_

    @functools.partial(
        pl.kernel,
        out_type=jax.ShapeDtypeStruct((NC_, NPAD_, D_), jnp.float32),
        mesh=_sc_mesh(),
        scratch_types=[
            pltpu.VMEM((nchunk, CHUNK_), jnp.int32),
            pltpu.VMEM((2, CHUNK_, D_), jnp.float32),
            pltpu.VMEM_SHARED((NPAD_, D_), jnp.float32),
            pltpu.SemaphoreType.DMA,
            pltpu.SemaphoreType.DMA,
            pltpu.SemaphoreType.DMA,
            pltpu.SemaphoreType.DMA,
        ],
    )
    def k(ea_hbm, idx_hbm, z_hbm, out_hbm, idx_v, buf, acc_sh, l0, l1, a0, a1):
        c = lax.axis_index("c")
        s = lax.axis_index("s")
        wid = s * NC_ + c
        rows_per_s = NPAD_ // NS_
        pltpu.sync_copy(z_hbm.at[c].at[pl.ds(s * rows_per_s, rows_per_s)],
                        acc_sh.at[pl.ds(s * rows_per_s, rows_per_s)])
        pltpu.sync_copy(idx_hbm.at[wid], idx_v)
        plsc.subcore_barrier()
        base = wid * epw
        lsems = (l0, l1)
        asems = (a0, a1)

        def start_l(j, slot):
            pltpu.async_copy(ea_hbm.at[pl.ds(base + j * CHUNK_, CHUNK_)],
                             buf.at[slot], lsems[slot])

        def wait_l(slot):
            pltpu.make_async_copy(ea_hbm.at[pl.ds(base, CHUNK_)],
                                  buf.at[slot], lsems[slot]).wait()

        def start_a(j, slot):
            pltpu.async_copy(buf.at[slot], acc_sh.at[idx_v.at[j]],
                             asems[slot], add=True)

        def wait_a(slot):
            pltpu.make_async_copy(buf.at[slot], acc_sh.at[idx_v.at[0]],
                                  asems[slot]).wait()

        start_l(0, 0)
        npairs = nchunk // 2

        def pair(k_, carry):
            j0 = 2 * k_

            @pl.when(k_ > 0)
            def _():
                wait_a(1)

            start_l(j0 + 1, 1)
            wait_l(0)
            start_a(j0, 0)
            wait_a(0)

            @pl.when(j0 + 2 < nchunk)
            def _():
                start_l(j0 + 2, 0)

            wait_l(1)
            start_a(j0 + 1, 1)
            return carry

        lax.fori_loop(0, npairs, pair, 0)
        wait_a(1)
        if nchunk % 2 == 1:
            wait_l(0)
            start_a(nchunk - 1, 0)
            wait_a(0)
        plsc.subcore_barrier()
        pltpu.sync_copy(acc_sh.at[pl.ds(s * rows_per_s, rows_per_s)],
                        out_hbm.at[c].at[pl.ds(s * rows_per_s, rows_per_s)])

    return k(ea, idx1, zinit)


def _prep_tc(x, w1a, w1b):
    BN = 2000
    grid = (N_ // BN,)

    def body(x_ref, wa_ref, wb_ref, p_ref, q_ref):
        xb = x_ref[...].astype(jnp.bfloat16)
        p_ref[...] = jnp.dot(xb, wa_ref[...], preferred_element_type=jnp.float32)
        q_ref[...] = jnp.dot(xb, wb_ref[...], preferred_element_type=jnp.float32)

    row = pl.BlockSpec((BN, D_), lambda i: (i, 0))
    w = pl.BlockSpec((D_, H_), lambda i: (0, 0))
    return pl.pallas_call(
        body, grid=grid,
        in_specs=[row, w, w],
        out_specs=[pl.BlockSpec((BN, H_), lambda i: (i, 0))] * 2,
        out_shape=[jax.ShapeDtypeStruct((N_, H_), jnp.float32)] * 2,
    )(x, w1a, w1b)


def _mlp_tail(h, w2, b2, w3cat, b3cat, g, bb):
    h = jnp.maximum(
        jnp.dot(h.astype(jnp.bfloat16), w2, preferred_element_type=jnp.float32)
        + b2, 0.0)
    t = jnp.dot(h.astype(jnp.bfloat16), w3cat,
                preferred_element_type=jnp.float32) + b3cat
    h = t[:, :H_]
    mu = t[:, H_:]
    d = h - mu
    var = jnp.mean(d * d, axis=-1, keepdims=True)
    return d * lax.rsqrt(var + 1e-5) * g + bb


def _ln_weights(w3, b3):
    w3m = jnp.tile(jnp.sum(w3, axis=1, keepdims=True) / H_, (1, H_))
    w3cat = jnp.concatenate([w3, w3m], axis=1).astype(jnp.bfloat16)
    b3cat = jnp.concatenate(
        [b3, jnp.full((H_,), jnp.mean(b3), jnp.float32)]).reshape(1, 2 * H_)
    return w3cat, b3cat


def _edge_mlp_tc(gsum, ea, w1c, b1, w2, b2, w3cat, b3cat, g, bb, e_tot):
    BE = 2560
    grid = (e_tot // BE,)

    def body(gs_ref, ea_ref, w1_ref, b1_ref, w2_ref, b2_ref,
             w3_ref, b3_ref, g_ref, bb_ref, out_ref):
        ea_b = ea_ref[...]
        h = (gs_ref[...] + b1_ref[...]
             + jnp.dot(ea_b.astype(jnp.bfloat16), w1_ref[...],
                       preferred_element_type=jnp.float32))
        h = jnp.maximum(h, 0.0)
        out_ref[...] = _mlp_tail(h, w2_ref[...], b2_ref[...], w3_ref[...],
                                 b3_ref[...], g_ref[...], bb_ref[...]) + ea_b

    row = pl.BlockSpec((BE, H_), lambda i: (i, 0))
    w = pl.BlockSpec((H_, H_), lambda i: (0, 0))
    wcat = pl.BlockSpec((H_, 2 * H_), lambda i: (0, 0))
    b = pl.BlockSpec((1, H_), lambda i: (0, 0))
    bcat = pl.BlockSpec((1, 2 * H_), lambda i: (0, 0))
    return pl.pallas_call(
        body, grid=grid,
        in_specs=[row, row, w, b, w, b, wcat, bcat, b, b],
        out_specs=pl.BlockSpec((BE, D_), lambda i: (i, 0)),
        out_shape=jax.ShapeDtypeStruct((e_tot, D_), jnp.float32),
    )(gsum, ea, w1c, b1.reshape(1, -1), w2, b2.reshape(1, -1),
      w3cat, b3cat, g.reshape(1, -1), bb.reshape(1, -1))


def _node_mlp_tc(x, parts, w1a, w1b, b1, w2, b2, w3cat, b3cat, g, bb):
    BN = 2000
    grid = (N_ // BN,)

    def body(x_ref, p_ref, w1a_ref, w1b_ref, b1_ref, w2_ref, b2_ref,
             w3_ref, b3_ref, g_ref, bb_ref, out_ref):
        xb = x_ref[...]
        agg = p_ref[0] + p_ref[1]
        h = (jnp.dot(xb.astype(jnp.bfloat16), w1a_ref[...],
                     preferred_element_type=jnp.float32)
             + jnp.dot(agg.astype(jnp.bfloat16), w1b_ref[...],
                       preferred_element_type=jnp.float32)
             + b1_ref[...])
        h = jnp.maximum(h, 0.0)
        out_ref[...] = _mlp_tail(h, w2_ref[...], b2_ref[...], w3_ref[...],
                                 b3_ref[...], g_ref[...], bb_ref[...]) + xb

    row = pl.BlockSpec((BN, D_), lambda i: (i, 0))
    prow = pl.BlockSpec((NC_, BN, D_), lambda i: (0, i, 0))
    w = pl.BlockSpec((D_, H_), lambda i: (0, 0))
    wcat = pl.BlockSpec((H_, 2 * H_), lambda i: (0, 0))
    b = pl.BlockSpec((1, H_), lambda i: (0, 0))
    bcat = pl.BlockSpec((1, 2 * H_), lambda i: (0, 0))
    return pl.pallas_call(
        body, grid=grid,
        in_specs=[row, prow, w, w, b, w, b, wcat, bcat, b, b],
        out_specs=row,
        out_shape=jax.ShapeDtypeStruct((N_, D_), jnp.float32),
    )(x, parts, w1a, w1b, b1.reshape(1, -1), w2, b2.reshape(1, -1),
      w3cat, b3cat, g.reshape(1, -1), bb.reshape(1, -1))


EA_ = 153600
EB_ = E_ - EA_


def kernel(x, edge_indices, edge_attrs, eW1, eb1, eW2, eb2, eW3, eb3, eg, ebb,
           nW1, nb1, nW2, nb2, nW3, nb3, ng, nbb):
    ei = edge_indices[0].astype(jnp.int32)
    idx0a = ei[0, :EA_].reshape(NW_, -1, CHUNK_)
    idx0b = ei[0, EA_:].reshape(NW_, -1, CHUNK_)
    idx1a = ei[1, :EA_].reshape(NW_, -1, CHUNK_)
    idx1b = ei[1, EA_:].reshape(NW_, -1, CHUNK_)
    ea_a = edge_attrs[0, :EA_]
    ea_b = edge_attrs[0, EA_:]
    zinit = jnp.zeros((NC_, NPAD_, D_), jnp.float32)

    eW1h = eW1.astype(jnp.bfloat16)
    eW2h = eW2.astype(jnp.bfloat16)
    nW1h = nW1.astype(jnp.bfloat16)
    nW2h = nW2.astype(jnp.bfloat16)

    for i in range(MP_):
        p_tab, q_tab = _prep_tc(x, eW1h[i, :D_], eW1h[i, D_:2 * D_])
        ew3cat, eb3cat = _ln_weights(eW3[i], eb3[i])
        gsum_a = _gather_add_sc(p_tab, q_tab, idx0a, idx1a, EA_)
        gsum_b = _gather_add_sc(p_tab, q_tab, idx0b, idx1b, EB_)
        ea_a = _edge_mlp_tc(gsum_a, ea_a, eW1h[i, 2 * D_:], eb1[i], eW2h[i],
                            eb2[i], ew3cat, eb3cat, eg[i], ebb[i], EA_)
        parts = _scatter_sc(ea_a, idx1a, zinit, EA_)
        ea_b = _edge_mlp_tc(gsum_b, ea_b, eW1h[i, 2 * D_:], eb1[i], eW2h[i],
                            eb2[i], ew3cat, eb3cat, eg[i], ebb[i], EB_)
        parts = _scatter_sc(ea_b, idx1b, parts, EB_)
        nw3cat, nb3cat = _ln_weights(nW3[i], nb3[i])
        x = _node_mlp_tc(x, parts, nW1h[i, :D_], nW1h[i, D_:], nb1[i],
                         nW2h[i], nb2[i], nw3cat, nb3cat, ng[i], nbb[i])
    return (x, jnp.concatenate([ea_a, ea_b], axis=0)[None])


# speedup vs baseline: 1.1885x; 1.0470x over previous
import functools

import numpy as np

import jax
import jax.numpy as jnp
from jax import lax
from jax.experimental import pallas as pl
from jax.experimental.pallas import tpu as pltpu
from jax.experimental.pallas import tpu_sc as plsc

MP_ = 2
N_ = 10000
E_ = 320000
D_ = 128
H_ = 128

NC_ = 2
NS_ = 16
NW_ = NC_ * NS_
EPW_ = E_ // NW_
CHUNK_ = 80
NCHUNK_ = EPW_ // CHUNK_
NPAD_ = 10240


def _sc_mesh():
    return plsc.VectorSubcoreMesh(core_axis_name="c", subcore_axis_name="s")


def _gather_add_sc(tab0, tab1, idx0, idx1, e_tot):
    epw = e_tot // NW_
    nchunk = epw // CHUNK_

    @functools.partial(
        pl.kernel,
        out_type=jax.ShapeDtypeStruct((e_tot, H_), jnp.float32),
        mesh=_sc_mesh(),
        scratch_types=[
            pltpu.VMEM((nchunk, CHUNK_), jnp.int32),
            pltpu.VMEM((nchunk, CHUNK_), jnp.int32),
            pltpu.VMEM((2, CHUNK_, H_), jnp.float32),
            pltpu.VMEM((2, CHUNK_, H_), jnp.float32),
            pltpu.VMEM((2, CHUNK_, H_), jnp.float32),
            pltpu.SemaphoreType.DMA,
            pltpu.SemaphoreType.DMA,
            pltpu.SemaphoreType.DMA,
            pltpu.SemaphoreType.DMA,
            pltpu.SemaphoreType.DMA,
            pltpu.SemaphoreType.DMA,
        ],
    )
    def k(tab0_hbm, tab1_hbm, idx0_hbm, idx1_hbm, out_hbm,
          idx0_v, idx1_v, bp, bq, bo, gp0, gp1, gq0, gq1, w0, w1):
        wid = lax.axis_index("s") * NC_ + lax.axis_index("c")
        pltpu.sync_copy(idx0_hbm.at[wid], idx0_v)
        pltpu.sync_copy(idx1_hbm.at[wid], idx1_v)
        base = wid * epw
        gsems = (gp0, gp1)
        qsems = (gq0, gq1)
        wsems = (w0, w1)

        def start_g(j, slot):
            pltpu.async_copy(tab0_hbm.at[idx0_v.at[j]], bp.at[slot], gsems[slot])
            pltpu.async_copy(tab1_hbm.at[idx1_v.at[j]], bq.at[slot], qsems[slot])

        def wait_g(slot):
            pltpu.make_async_copy(
                tab0_hbm.at[idx0_v.at[0]], bp.at[slot], gsems[slot]).wait()
            pltpu.make_async_copy(
                tab1_hbm.at[idx1_v.at[0]], bq.at[slot], qsems[slot]).wait()

        def start_w(j, slot):
            pltpu.async_copy(
                bo.at[slot], out_hbm.at[pl.ds(base + j * CHUNK_, CHUNK_)],
                wsems[slot])

        def wait_w(slot):
            pltpu.make_async_copy(
                bo.at[slot], out_hbm.at[pl.ds(base, CHUNK_)], wsems[slot]).wait()

        def compute(slot):
            bp_s, bq_s, bo_s = bp.at[slot], bq.at[slot], bo.at[slot]

            def row(r, carry):
                for c in range(8):
                    bo_s[r, pl.ds(16 * c, 16)] = (
                        bp_s[r, pl.ds(16 * c, 16)] + bq_s[r, pl.ds(16 * c, 16)])
                return carry

            lax.fori_loop(0, CHUNK_, row, 0)

        start_g(0, 0)
        npairs = nchunk // 2

        def pair(k_, carry):
            j0 = 2 * k_
            start_g(j0 + 1, 1)
            wait_g(0)

            @pl.when(k_ > 0)
            def _():
                wait_w(0)

            compute(0)
            start_w(j0, 0)

            @pl.when(j0 + 2 < nchunk)
            def _():
                start_g(j0 + 2, 0)

            wait_g(1)

            @pl.when(k_ > 0)
            def _():
                wait_w(1)

            compute(1)
            start_w(j0 + 1, 1)
            return carry

        lax.fori_loop(0, npairs, pair, 0)
        if nchunk % 2 == 1:
            wait_g(0)
            wait_w(0)
            compute(0)
            start_w(nchunk - 1, 0)
        wait_w(0)
        wait_w(1)

    return k(tab0, tab1, idx0, idx1)


def _scatter_sc(ea, idx1, zinit, e_tot):
    epw = e_tot // NW_
    nchunk = epw // CHUNK_

    @functools.partial(
        pl.kernel,
        out_type=jax.ShapeDtypeStruct((NC_, NPAD_, D_), jnp.float32),
        mesh=_sc_mesh(),
        scratch_types=[
            pltpu.VMEM((nchunk, CHUNK_), jnp.int32),
            pltpu.VMEM((2, CHUNK_, D_), jnp.float32),
            pltpu.VMEM_SHARED((NPAD_, D_), jnp.float32),
            pltpu.SemaphoreType.DMA,
            pltpu.SemaphoreType.DMA,
            pltpu.SemaphoreType.DMA,
            pltpu.SemaphoreType.DMA,
        ],
    )
    def k(ea_hbm, idx_hbm, z_hbm, out_hbm, idx_v, buf, acc_sh, l0, l1, a0, a1):
        c = lax.axis_index("c")
        s = lax.axis_index("s")
        wid = s * NC_ + c
        rows_per_s = NPAD_ // NS_
        pltpu.sync_copy(z_hbm.at[c].at[pl.ds(s * rows_per_s, rows_per_s)],
                        acc_sh.at[pl.ds(s * rows_per_s, rows_per_s)])
        pltpu.sync_copy(idx_hbm.at[wid], idx_v)
        plsc.subcore_barrier()
        base = wid * epw
        lsems = (l0, l1)
        asems = (a0, a1)

        def start_l(j, slot):
            pltpu.async_copy(ea_hbm.at[pl.ds(base + j * CHUNK_, CHUNK_)],
                             buf.at[slot], lsems[slot])

        def wait_l(slot):
            pltpu.make_async_copy(ea_hbm.at[pl.ds(base, CHUNK_)],
                                  buf.at[slot], lsems[slot]).wait()

        def start_a(j, slot):
            pltpu.async_copy(buf.at[slot], acc_sh.at[idx_v.at[j]],
                             asems[slot], add=True)

        def wait_a(slot):
            pltpu.make_async_copy(buf.at[slot], acc_sh.at[idx_v.at[0]],
                                  asems[slot]).wait()

        start_l(0, 0)
        npairs = nchunk // 2

        def pair(k_, carry):
            j0 = 2 * k_

            @pl.when(k_ > 0)
            def _():
                wait_a(1)

            start_l(j0 + 1, 1)
            wait_l(0)
            start_a(j0, 0)
            wait_a(0)

            @pl.when(j0 + 2 < nchunk)
            def _():
                start_l(j0 + 2, 0)

            wait_l(1)
            start_a(j0 + 1, 1)
            return carry

        lax.fori_loop(0, npairs, pair, 0)
        wait_a(1)
        if nchunk % 2 == 1:
            wait_l(0)
            start_a(nchunk - 1, 0)
            wait_a(0)
        plsc.subcore_barrier()
        pltpu.sync_copy(acc_sh.at[pl.ds(s * rows_per_s, rows_per_s)],
                        out_hbm.at[c].at[pl.ds(s * rows_per_s, rows_per_s)])

    return k(ea, idx1, zinit)


def _prep_tc(x, w1a, w1b):
    BN = 2000
    grid = (N_ // BN,)

    def body(x_ref, wa_ref, wb_ref, p_ref, q_ref):
        xb = x_ref[...].astype(jnp.bfloat16)
        p_ref[...] = jnp.dot(xb, wa_ref[...], preferred_element_type=jnp.float32)
        q_ref[...] = jnp.dot(xb, wb_ref[...], preferred_element_type=jnp.float32)

    row = pl.BlockSpec((BN, D_), lambda i: (i, 0))
    w = pl.BlockSpec((D_, H_), lambda i: (0, 0))
    return pl.pallas_call(
        body, grid=grid,
        in_specs=[row, w, w],
        out_specs=[pl.BlockSpec((BN, H_), lambda i: (i, 0))] * 2,
        out_shape=[jax.ShapeDtypeStruct((N_, H_), jnp.float32)] * 2,
    )(x, w1a, w1b)


def _mlp_tail(h, w2, b2, w3cat, b3cat, g, bb):
    h = jnp.maximum(
        jnp.dot(h.astype(jnp.bfloat16), w2, preferred_element_type=jnp.float32)
        + b2, 0.0)
    t = jnp.dot(h.astype(jnp.bfloat16), w3cat,
                preferred_element_type=jnp.float32) + b3cat
    h = t[:, :H_]
    mu = t[:, H_:]
    d = h - mu
    var = jnp.mean(d * d, axis=-1, keepdims=True)
    return d * lax.rsqrt(var + 1e-5) * g + bb


def _ln_weights(w3, b3):
    w3m = jnp.tile(jnp.sum(w3, axis=1, keepdims=True) / H_, (1, H_))
    w3cat = jnp.concatenate([w3, w3m], axis=1).astype(jnp.bfloat16)
    b3cat = jnp.concatenate(
        [b3, jnp.full((H_,), jnp.mean(b3), jnp.float32)]).reshape(1, 2 * H_)
    return w3cat, b3cat


def _edge_mlp_tc(gsum, ea, w1c, b1, w2, b2, w3cat, b3cat, g, bb, e_tot):
    BE = 6400
    grid = (e_tot // BE,)

    def body(gs_ref, ea_ref, w1_ref, b1_ref, w2_ref, b2_ref,
             w3_ref, b3_ref, g_ref, bb_ref, out_ref):
        ea_b = ea_ref[...]
        h = (gs_ref[...] + b1_ref[...]
             + jnp.dot(ea_b.astype(jnp.bfloat16), w1_ref[...],
                       preferred_element_type=jnp.float32))
        h = jnp.maximum(h, 0.0)
        out_ref[...] = _mlp_tail(h, w2_ref[...], b2_ref[...], w3_ref[...],
                                 b3_ref[...], g_ref[...], bb_ref[...]) + ea_b

    row = pl.BlockSpec((BE, H_), lambda i: (i, 0))
    w = pl.BlockSpec((H_, H_), lambda i: (0, 0))
    wcat = pl.BlockSpec((H_, 2 * H_), lambda i: (0, 0))
    b = pl.BlockSpec((1, H_), lambda i: (0, 0))
    bcat = pl.BlockSpec((1, 2 * H_), lambda i: (0, 0))
    return pl.pallas_call(
        body, grid=grid,
        in_specs=[row, row, w, b, w, b, wcat, bcat, b, b],
        out_specs=pl.BlockSpec((BE, D_), lambda i: (i, 0)),
        out_shape=jax.ShapeDtypeStruct((e_tot, D_), jnp.float32),
    )(gsum, ea, w1c, b1.reshape(1, -1), w2, b2.reshape(1, -1),
      w3cat, b3cat, g.reshape(1, -1), bb.reshape(1, -1))


def _node_mlp_tc(x, parts, w1a, w1b, b1, w2, b2, w3cat, b3cat, g, bb):
    BN = 2000
    grid = (N_ // BN,)

    def body(x_ref, p_ref, w1a_ref, w1b_ref, b1_ref, w2_ref, b2_ref,
             w3_ref, b3_ref, g_ref, bb_ref, out_ref):
        xb = x_ref[...]
        agg = p_ref[0] + p_ref[1]
        h = (jnp.dot(xb.astype(jnp.bfloat16), w1a_ref[...],
                     preferred_element_type=jnp.float32)
             + jnp.dot(agg.astype(jnp.bfloat16), w1b_ref[...],
                       preferred_element_type=jnp.float32)
             + b1_ref[...])
        h = jnp.maximum(h, 0.0)
        out_ref[...] = _mlp_tail(h, w2_ref[...], b2_ref[...], w3_ref[...],
                                 b3_ref[...], g_ref[...], bb_ref[...]) + xb

    row = pl.BlockSpec((BN, D_), lambda i: (i, 0))
    prow = pl.BlockSpec((NC_, BN, D_), lambda i: (0, i, 0))
    w = pl.BlockSpec((D_, H_), lambda i: (0, 0))
    wcat = pl.BlockSpec((H_, 2 * H_), lambda i: (0, 0))
    b = pl.BlockSpec((1, H_), lambda i: (0, 0))
    bcat = pl.BlockSpec((1, 2 * H_), lambda i: (0, 0))
    return pl.pallas_call(
        body, grid=grid,
        in_specs=[row, prow, w, w, b, w, b, wcat, bcat, b, b],
        out_specs=row,
        out_shape=jax.ShapeDtypeStruct((N_, D_), jnp.float32),
    )(x, parts, w1a, w1b, b1.reshape(1, -1), w2, b2.reshape(1, -1),
      w3cat, b3cat, g.reshape(1, -1), bb.reshape(1, -1))


EA_ = 153600
EB_ = E_ - EA_


def kernel(x, edge_indices, edge_attrs, eW1, eb1, eW2, eb2, eW3, eb3, eg, ebb,
           nW1, nb1, nW2, nb2, nW3, nb3, ng, nbb):
    ei = edge_indices[0].astype(jnp.int32)
    idx0a = ei[0, :EA_].reshape(NW_, -1, CHUNK_)
    idx0b = ei[0, EA_:].reshape(NW_, -1, CHUNK_)
    idx1a = ei[1, :EA_].reshape(NW_, -1, CHUNK_)
    idx1b = ei[1, EA_:].reshape(NW_, -1, CHUNK_)
    ea_a = edge_attrs[0, :EA_]
    ea_b = edge_attrs[0, EA_:]
    zinit = jnp.zeros((NC_, NPAD_, D_), jnp.float32)

    eW1h = eW1.astype(jnp.bfloat16)
    eW2h = eW2.astype(jnp.bfloat16)
    nW1h = nW1.astype(jnp.bfloat16)
    nW2h = nW2.astype(jnp.bfloat16)

    for i in range(MP_):
        p_tab, q_tab = _prep_tc(x, eW1h[i, :D_], eW1h[i, D_:2 * D_])
        ew3cat, eb3cat = _ln_weights(eW3[i], eb3[i])
        gsum_a = _gather_add_sc(p_tab, q_tab, idx0a, idx1a, EA_)
        gsum_b = _gather_add_sc(p_tab, q_tab, idx0b, idx1b, EB_)
        ea_a = _edge_mlp_tc(gsum_a, ea_a, eW1h[i, 2 * D_:], eb1[i], eW2h[i],
                            eb2[i], ew3cat, eb3cat, eg[i], ebb[i], EA_)
        parts = _scatter_sc(ea_a, idx1a, zinit, EA_)
        ea_b = _edge_mlp_tc(gsum_b, ea_b, eW1h[i, 2 * D_:], eb1[i], eW2h[i],
                            eb2[i], ew3cat, eb3cat, eg[i], ebb[i], EB_)
        parts = _scatter_sc(ea_b, idx1b, parts, EB_)
        nw3cat, nb3cat = _ln_weights(nW3[i], nb3[i])
        x = _node_mlp_tc(x, parts, nW1h[i, :D_], nW1h[i, D_:], nb1[i],
                         nW2h[i], nb2[i], nw3cat, nb3cat, ng[i], nbb[i])
    return (x, jnp.concatenate([ea_a, ea_b], axis=0)[None])


# speedup vs baseline: 1.1940x; 1.0047x over previous
import functools

import numpy as np

import jax
import jax.numpy as jnp
from jax import lax
from jax.experimental import pallas as pl
from jax.experimental.pallas import tpu as pltpu
from jax.experimental.pallas import tpu_sc as plsc

MP_ = 2
N_ = 10000
E_ = 320000
D_ = 128
H_ = 128

NC_ = 2
NS_ = 16
NW_ = NC_ * NS_
EPW_ = E_ // NW_
CHUNK_ = 80
NCHUNK_ = EPW_ // CHUNK_
NPAD_ = 10240


def _sc_mesh():
    return plsc.VectorSubcoreMesh(core_axis_name="c", subcore_axis_name="s")


def _gather_add_sc(tab0, tab1, idx0, idx1, e_tot):
    epw = e_tot // NW_
    nchunk = epw // CHUNK_

    @functools.partial(
        pl.kernel,
        out_type=jax.ShapeDtypeStruct((e_tot, H_), jnp.float32),
        mesh=_sc_mesh(),
        scratch_types=[
            pltpu.VMEM((nchunk, CHUNK_), jnp.int32),
            pltpu.VMEM((nchunk, CHUNK_), jnp.int32),
            pltpu.VMEM((2, CHUNK_, H_), jnp.float32),
            pltpu.VMEM((2, CHUNK_, H_), jnp.float32),
            pltpu.VMEM((2, CHUNK_, H_), jnp.float32),
            pltpu.SemaphoreType.DMA,
            pltpu.SemaphoreType.DMA,
            pltpu.SemaphoreType.DMA,
            pltpu.SemaphoreType.DMA,
            pltpu.SemaphoreType.DMA,
            pltpu.SemaphoreType.DMA,
        ],
    )
    def k(tab0_hbm, tab1_hbm, idx0_hbm, idx1_hbm, out_hbm,
          idx0_v, idx1_v, bp, bq, bo, gp0, gp1, gq0, gq1, w0, w1):
        wid = lax.axis_index("s") * NC_ + lax.axis_index("c")
        pltpu.sync_copy(idx0_hbm.at[wid], idx0_v)
        pltpu.sync_copy(idx1_hbm.at[wid], idx1_v)
        base = wid * epw
        gsems = (gp0, gp1)
        qsems = (gq0, gq1)
        wsems = (w0, w1)

        def start_g(j, slot):
            pltpu.async_copy(tab0_hbm.at[idx0_v.at[j]], bp.at[slot], gsems[slot])
            pltpu.async_copy(tab1_hbm.at[idx1_v.at[j]], bq.at[slot], qsems[slot])

        def wait_g(slot):
            pltpu.make_async_copy(
                tab0_hbm.at[idx0_v.at[0]], bp.at[slot], gsems[slot]).wait()
            pltpu.make_async_copy(
                tab1_hbm.at[idx1_v.at[0]], bq.at[slot], qsems[slot]).wait()

        def start_w(j, slot):
            pltpu.async_copy(
                bo.at[slot], out_hbm.at[pl.ds(base + j * CHUNK_, CHUNK_)],
                wsems[slot])

        def wait_w(slot):
            pltpu.make_async_copy(
                bo.at[slot], out_hbm.at[pl.ds(base, CHUNK_)], wsems[slot]).wait()

        def compute(slot):
            bp_s, bq_s, bo_s = bp.at[slot], bq.at[slot], bo.at[slot]

            def row(r, carry):
                for c in range(8):
                    bo_s[r, pl.ds(16 * c, 16)] = (
                        bp_s[r, pl.ds(16 * c, 16)] + bq_s[r, pl.ds(16 * c, 16)])
                return carry

            lax.fori_loop(0, CHUNK_, row, 0)

        start_g(0, 0)
        npairs = nchunk // 2

        def pair(k_, carry):
            j0 = 2 * k_
            start_g(j0 + 1, 1)
            wait_g(0)

            @pl.when(k_ > 0)
            def _():
                wait_w(0)

            compute(0)
            start_w(j0, 0)

            @pl.when(j0 + 2 < nchunk)
            def _():
                start_g(j0 + 2, 0)

            wait_g(1)

            @pl.when(k_ > 0)
            def _():
                wait_w(1)

            compute(1)
            start_w(j0 + 1, 1)
            return carry

        lax.fori_loop(0, npairs, pair, 0)
        if nchunk % 2 == 1:
            wait_g(0)
            wait_w(0)
            compute(0)
            start_w(nchunk - 1, 0)
        wait_w(0)
        wait_w(1)

    return k(tab0, tab1, idx0, idx1)


def _scatter_sc(ea, idx1, zinit, e_tot):
    epw = e_tot // NW_
    nchunk = epw // CHUNK_

    @functools.partial(
        pl.kernel,
        out_type=jax.ShapeDtypeStruct((NC_, NPAD_, D_), jnp.float32),
        mesh=_sc_mesh(),
        scratch_types=[
            pltpu.VMEM((nchunk, CHUNK_), jnp.int32),
            pltpu.VMEM((2, CHUNK_, D_), jnp.float32),
            pltpu.VMEM_SHARED((NPAD_, D_), jnp.float32),
            pltpu.SemaphoreType.DMA,
            pltpu.SemaphoreType.DMA,
            pltpu.SemaphoreType.DMA,
            pltpu.SemaphoreType.DMA,
        ],
    )
    def k(ea_hbm, idx_hbm, z_hbm, out_hbm, idx_v, buf, acc_sh, l0, l1, a0, a1):
        c = lax.axis_index("c")
        s = lax.axis_index("s")
        wid = s * NC_ + c
        rows_per_s = NPAD_ // NS_
        pltpu.sync_copy(z_hbm.at[c].at[pl.ds(s * rows_per_s, rows_per_s)],
                        acc_sh.at[pl.ds(s * rows_per_s, rows_per_s)])
        pltpu.sync_copy(idx_hbm.at[wid], idx_v)
        plsc.subcore_barrier()
        base = wid * epw
        lsems = (l0, l1)
        asems = (a0, a1)

        def start_l(j, slot):
            pltpu.async_copy(ea_hbm.at[pl.ds(base + j * CHUNK_, CHUNK_)],
                             buf.at[slot], lsems[slot])

        def wait_l(slot):
            pltpu.make_async_copy(ea_hbm.at[pl.ds(base, CHUNK_)],
                                  buf.at[slot], lsems[slot]).wait()

        def start_a(j, slot):
            pltpu.async_copy(buf.at[slot], acc_sh.at[idx_v.at[j]],
                             asems[slot], add=True)

        def wait_a(slot):
            pltpu.make_async_copy(buf.at[slot], acc_sh.at[idx_v.at[0]],
                                  asems[slot]).wait()

        start_l(0, 0)
        npairs = nchunk // 2

        def pair(k_, carry):
            j0 = 2 * k_

            @pl.when(k_ > 0)
            def _():
                wait_a(1)

            start_l(j0 + 1, 1)
            wait_l(0)
            start_a(j0, 0)
            wait_a(0)

            @pl.when(j0 + 2 < nchunk)
            def _():
                start_l(j0 + 2, 0)

            wait_l(1)
            start_a(j0 + 1, 1)
            return carry

        lax.fori_loop(0, npairs, pair, 0)
        wait_a(1)
        if nchunk % 2 == 1:
            wait_l(0)
            start_a(nchunk - 1, 0)
            wait_a(0)
        plsc.subcore_barrier()
        pltpu.sync_copy(acc_sh.at[pl.ds(s * rows_per_s, rows_per_s)],
                        out_hbm.at[c].at[pl.ds(s * rows_per_s, rows_per_s)])

    return k(ea, idx1, zinit)


def _prep_tc(x, w1a, w1b):
    BN = 2000
    grid = (N_ // BN,)

    def body(x_ref, wa_ref, wb_ref, p_ref, q_ref):
        xb = x_ref[...].astype(jnp.bfloat16)
        p_ref[...] = jnp.dot(xb, wa_ref[...], preferred_element_type=jnp.float32)
        q_ref[...] = jnp.dot(xb, wb_ref[...], preferred_element_type=jnp.float32)

    row = pl.BlockSpec((BN, D_), lambda i: (i, 0))
    w = pl.BlockSpec((D_, H_), lambda i: (0, 0))
    return pl.pallas_call(
        body, grid=grid,
        in_specs=[row, w, w],
        out_specs=[pl.BlockSpec((BN, H_), lambda i: (i, 0))] * 2,
        out_shape=[jax.ShapeDtypeStruct((N_, H_), jnp.float32)] * 2,
    )(x, w1a, w1b)


def _mlp_tail(h, w2, b2, w3cat, b3cat, g, bb):
    h = jnp.maximum(
        jnp.dot(h.astype(jnp.bfloat16), w2, preferred_element_type=jnp.float32)
        + b2, 0.0)
    t = jnp.dot(h.astype(jnp.bfloat16), w3cat,
                preferred_element_type=jnp.float32) + b3cat
    h = t[:, :H_]
    mu = t[:, H_:]
    d = h - mu
    var = jnp.mean(d * d, axis=-1, keepdims=True)
    return d * lax.rsqrt(var + 1e-5) * g + bb


def _ln_weights(w3, b3):
    w3m = jnp.tile(jnp.sum(w3, axis=1, keepdims=True) / H_, (1, H_))
    w3cat = jnp.concatenate([w3, w3m], axis=1).astype(jnp.bfloat16)
    b3cat = jnp.concatenate(
        [b3, jnp.full((H_,), jnp.mean(b3), jnp.float32)]).reshape(1, 2 * H_)
    return w3cat, b3cat


def _edge_mlp_tc(gsum, ea, w1c, b1, w2, b2, w3cat, b3cat, g, bb, e_tot):
    BE = 12800
    grid = (e_tot // BE,)

    def body(gs_ref, ea_ref, w1_ref, b1_ref, w2_ref, b2_ref,
             w3_ref, b3_ref, g_ref, bb_ref, out_ref):
        ea_b = ea_ref[...]
        h = (gs_ref[...] + b1_ref[...]
             + jnp.dot(ea_b.astype(jnp.bfloat16), w1_ref[...],
                       preferred_element_type=jnp.float32))
        h = jnp.maximum(h, 0.0)
        out_ref[...] = _mlp_tail(h, w2_ref[...], b2_ref[...], w3_ref[...],
                                 b3_ref[...], g_ref[...], bb_ref[...]) + ea_b

    row = pl.BlockSpec((BE, H_), lambda i: (i, 0))
    w = pl.BlockSpec((H_, H_), lambda i: (0, 0))
    wcat = pl.BlockSpec((H_, 2 * H_), lambda i: (0, 0))
    b = pl.BlockSpec((1, H_), lambda i: (0, 0))
    bcat = pl.BlockSpec((1, 2 * H_), lambda i: (0, 0))
    return pl.pallas_call(
        body, grid=grid,
        in_specs=[row, row, w, b, w, b, wcat, bcat, b, b],
        out_specs=pl.BlockSpec((BE, D_), lambda i: (i, 0)),
        out_shape=jax.ShapeDtypeStruct((e_tot, D_), jnp.float32),
    )(gsum, ea, w1c, b1.reshape(1, -1), w2, b2.reshape(1, -1),
      w3cat, b3cat, g.reshape(1, -1), bb.reshape(1, -1))


def _node_mlp_tc(x, parts, w1a, w1b, b1, w2, b2, w3cat, b3cat, g, bb):
    BN = 2000
    grid = (N_ // BN,)

    def body(x_ref, p_ref, w1a_ref, w1b_ref, b1_ref, w2_ref, b2_ref,
             w3_ref, b3_ref, g_ref, bb_ref, out_ref):
        xb = x_ref[...]
        agg = p_ref[0] + p_ref[1]
        h = (jnp.dot(xb.astype(jnp.bfloat16), w1a_ref[...],
                     preferred_element_type=jnp.float32)
             + jnp.dot(agg.astype(jnp.bfloat16), w1b_ref[...],
                       preferred_element_type=jnp.float32)
             + b1_ref[...])
        h = jnp.maximum(h, 0.0)
        out_ref[...] = _mlp_tail(h, w2_ref[...], b2_ref[...], w3_ref[...],
                                 b3_ref[...], g_ref[...], bb_ref[...]) + xb

    row = pl.BlockSpec((BN, D_), lambda i: (i, 0))
    prow = pl.BlockSpec((NC_, BN, D_), lambda i: (0, i, 0))
    w = pl.BlockSpec((D_, H_), lambda i: (0, 0))
    wcat = pl.BlockSpec((H_, 2 * H_), lambda i: (0, 0))
    b = pl.BlockSpec((1, H_), lambda i: (0, 0))
    bcat = pl.BlockSpec((1, 2 * H_), lambda i: (0, 0))
    return pl.pallas_call(
        body, grid=grid,
        in_specs=[row, prow, w, w, b, w, b, wcat, bcat, b, b],
        out_specs=row,
        out_shape=jax.ShapeDtypeStruct((N_, D_), jnp.float32),
    )(x, parts, w1a, w1b, b1.reshape(1, -1), w2, b2.reshape(1, -1),
      w3cat, b3cat, g.reshape(1, -1), bb.reshape(1, -1))


EA_ = 153600
EB_ = E_ - EA_


def kernel(x, edge_indices, edge_attrs, eW1, eb1, eW2, eb2, eW3, eb3, eg, ebb,
           nW1, nb1, nW2, nb2, nW3, nb3, ng, nbb):
    ei = edge_indices[0].astype(jnp.int32)
    idx0a = ei[0, :EA_].reshape(NW_, -1, CHUNK_)
    idx0b = ei[0, EA_:].reshape(NW_, -1, CHUNK_)
    idx1a = ei[1, :EA_].reshape(NW_, -1, CHUNK_)
    idx1b = ei[1, EA_:].reshape(NW_, -1, CHUNK_)
    ea_a = edge_attrs[0, :EA_]
    ea_b = edge_attrs[0, EA_:]
    zinit = jnp.zeros((NC_, NPAD_, D_), jnp.float32)

    eW1h = eW1.astype(jnp.bfloat16)
    eW2h = eW2.astype(jnp.bfloat16)
    nW1h = nW1.astype(jnp.bfloat16)
    nW2h = nW2.astype(jnp.bfloat16)

    for i in range(MP_):
        p_tab, q_tab = _prep_tc(x, eW1h[i, :D_], eW1h[i, D_:2 * D_])
        ew3cat, eb3cat = _ln_weights(eW3[i], eb3[i])
        gsum_a = _gather_add_sc(p_tab, q_tab, idx0a, idx1a, EA_)
        gsum_b = _gather_add_sc(p_tab, q_tab, idx0b, idx1b, EB_)
        ea_a = _edge_mlp_tc(gsum_a, ea_a, eW1h[i, 2 * D_:], eb1[i], eW2h[i],
                            eb2[i], ew3cat, eb3cat, eg[i], ebb[i], EA_)
        parts = _scatter_sc(ea_a, idx1a, zinit, EA_)
        ea_b = _edge_mlp_tc(gsum_b, ea_b, eW1h[i, 2 * D_:], eb1[i], eW2h[i],
                            eb2[i], ew3cat, eb3cat, eg[i], ebb[i], EB_)
        parts = _scatter_sc(ea_b, idx1b, parts, EB_)
        nw3cat, nb3cat = _ln_weights(nW3[i], nb3[i])
        x = _node_mlp_tc(x, parts, nW1h[i, :D_], nW1h[i, D_:], nb1[i],
                         nW2h[i], nb2[i], nw3cat, nb3cat, ng[i], nbb[i])
    return (x, jnp.concatenate([ea_a, ea_b], axis=0)[None])
